# Initial kernel scaffold; baseline (speedup 1.0000x reference)
import jax
import jax.numpy as jnp
from jax.experimental import pallas as pl


def kernel(x, cluster_labels, table):
    raise NotImplementedError("write your pallas kernel here")


# speedup vs baseline: 2.4511x; 2.4511x over previous
import functools

import jax
import jax.numpy as jnp
from jax import lax
from jax.experimental import pallas as pl
from jax.experimental.pallas import tpu as pltpu
from jax.experimental.pallas import tpu_sc as plsc

_NC = 2
_NS = 16
_NW = _NC * _NS
_LANES = 16
_GSUB = 128


def _sc_gather_add(x_flat, labels_flat, table, *, rows_per_chunk=384):
    n, d = x_flat.shape
    rows_per_w = n // _NW
    assert rows_per_w * _NW == n
    nchunks = rows_per_w // rows_per_chunk
    assert nchunks * rows_per_chunk == rows_per_w
    assert rows_per_chunk % _GSUB == 0 and rows_per_chunk % 8 == 0
    r = rows_per_chunk

    mesh = plsc.VectorSubcoreMesh(
        core_axis_name="c", subcore_axis_name="s",
        num_cores=_NC, num_subcores=_NS)

    @functools.partial(
        pl.kernel,
        out_type=jax.ShapeDtypeStruct((n, d), jnp.float32),
        mesh=mesh,
        compiler_params=pltpu.CompilerParams(use_tc_tiling_on_sc=False),
        scratch_types=[
            pltpu.VMEM((r,), jnp.int32),
            pltpu.VMEM((r, d), jnp.float32),
            pltpu.VMEM((r, d), jnp.float32),
            pltpu.SemaphoreType.DMA,
        ],
    )
    def k(x_hbm, idx_hbm, table_hbm, out_hbm, idx_v, rows_v, x_v, sem):
        wid = lax.axis_index("s") * _NC + lax.axis_index("c")
        wbase = wid * rows_per_w

        def chunk(c, carry):
            base = wbase + c * r
            pltpu.sync_copy(idx_hbm.at[pl.ds(base, r)], idx_v)
            descs = []
            for j in range(r // _GSUB):
                descs.append(pltpu.async_copy(
                    table_hbm.at[idx_v.at[pl.ds(j * _GSUB, _GSUB)]],
                    rows_v.at[pl.ds(j * _GSUB, _GSUB)],
                    sem))
            pltpu.sync_copy(x_hbm.at[pl.ds(base, r)], x_v)
            for dsc in descs:
                dsc.wait()

            def add_row(i, carry2):
                for j in range(d // _LANES):
                    sl = pl.ds(j * _LANES, _LANES)
                    rows_v[i, sl] = rows_v[i, sl] + x_v[i, sl]
                return carry2

            lax.fori_loop(0, r, add_row, 0)
            pltpu.sync_copy(rows_v, out_hbm.at[pl.ds(base, r)])
            return carry

        lax.fori_loop(0, nchunks, chunk, 0)

    return k(x_flat, labels_flat, table)


def kernel(x, cluster_labels, table):
    b, lp1, d = x.shape
    zeros_col = jnp.zeros((b, 1), dtype=cluster_labels.dtype)
    labels = jnp.concatenate([zeros_col, cluster_labels], axis=1).reshape(-1)
    out = _sc_gather_add(x.reshape(-1, d), labels, table)
    return out.reshape(b, lp1, d)


# speedup vs baseline: 2.5511x; 1.0408x over previous
import functools

import jax
import jax.numpy as jnp
from jax import lax
from jax.experimental import pallas as pl
from jax.experimental.pallas import tpu as pltpu
from jax.experimental.pallas import tpu_sc as plsc

_NC = 2
_NS = 16
_NW = _NC * _NS
_LANES = 16
_GSUB = 128


def _sc_gather_add(x_flat, labels_flat, table, *, rows_per_chunk=384):
    n, d = x_flat.shape
    rows_per_w = n // _NW
    assert rows_per_w * _NW == n
    nchunks = rows_per_w // rows_per_chunk
    assert nchunks * rows_per_chunk == rows_per_w
    assert rows_per_chunk % _GSUB == 0 and rows_per_chunk % 8 == 0
    r = rows_per_chunk

    mesh = plsc.VectorSubcoreMesh(
        core_axis_name="c", subcore_axis_name="s",
        num_cores=_NC, num_subcores=_NS)

    @functools.partial(
        pl.kernel,
        out_type=jax.ShapeDtypeStruct((n, d), jnp.float32),
        mesh=mesh,
        compiler_params=pltpu.CompilerParams(use_tc_tiling_on_sc=False),
        scratch_types=[
            pltpu.VMEM((r,), jnp.int32),
            pltpu.VMEM((r, d), jnp.float32),
            pltpu.VMEM((r, d), jnp.float32),
            pltpu.SemaphoreType.DMA,
        ],
    )
    def k(x_hbm, idx_hbm, table_hbm, out_hbm, idx_v, rows_v, x_v, sem):
        wid = lax.axis_index("s") * _NC + lax.axis_index("c")
        wbase = wid * rows_per_w

        def chunk(c, carry):
            base = wbase + c * r
            pltpu.sync_copy(idx_hbm.at[pl.ds(base, r)], idx_v)
            pltpu.sync_copy(x_hbm.at[pl.ds(base, r)], rows_v)
            descs = []
            for j in range(r // _GSUB):
                descs.append(pltpu.async_copy(
                    table_hbm.at[idx_v.at[pl.ds(j * _GSUB, _GSUB)]],
                    rows_v.at[pl.ds(j * _GSUB, _GSUB)],
                    sem, add=True))
            for dsc in descs:
                dsc.wait()
            pltpu.sync_copy(rows_v, out_hbm.at[pl.ds(base, r)])
            return carry

        lax.fori_loop(0, nchunks, chunk, 0)

    return k(x_flat, labels_flat, table)


def kernel(x, cluster_labels, table):
    b, lp1, d = x.shape
    zeros_col = jnp.zeros((b, 1), dtype=cluster_labels.dtype)
    labels = jnp.concatenate([zeros_col, cluster_labels], axis=1).reshape(-1)
    out = _sc_gather_add(x.reshape(-1, d), labels, table)
    return out.reshape(b, lp1, d)


# speedup vs baseline: 2.5920x; 1.0160x over previous
import functools

import jax
import jax.numpy as jnp
from jax import lax
from jax.experimental import pallas as pl
from jax.experimental.pallas import tpu as pltpu
from jax.experimental.pallas import tpu_sc as plsc

_NC = 2
_NS = 16
_NW = _NC * _NS


def _sc_gather_add(x_flat, labels_flat, table, *, r=96, nbuf=4, la=2):
    n, d = x_flat.shape
    rows_per_w = n // _NW
    assert rows_per_w * _NW == n
    nchunks = rows_per_w // r
    assert nchunks * r == rows_per_w
    rounds = nchunks // nbuf
    assert rounds * nbuf == nchunks and rounds >= 3
    assert r <= 128 and r % 8 == 0
    assert 1 <= la <= nbuf - 1

    mesh = plsc.VectorSubcoreMesh(
        core_axis_name="c", subcore_axis_name="s",
        num_cores=_NC, num_subcores=_NS)

    @functools.partial(
        pl.kernel,
        out_type=jax.ShapeDtypeStruct((n, d), jnp.float32),
        mesh=mesh,
        compiler_params=pltpu.CompilerParams(use_tc_tiling_on_sc=False),
        scratch_types=(
            [pltpu.VMEM((r,), jnp.int32) for _ in range(nbuf)]
            + [pltpu.VMEM((r, d), jnp.float32) for _ in range(nbuf)]
            + [pltpu.SemaphoreType.DMA for _ in range(3 * nbuf)]
        ),
    )
    def k(x_hbm, idx_hbm, table_hbm, out_hbm, *scr):
        idx_v = scr[:nbuf]
        rows_v = scr[nbuf:2 * nbuf]
        s_in = scr[2 * nbuf:3 * nbuf]
        s_g = scr[3 * nbuf:4 * nbuf]
        s_o = scr[4 * nbuf:5 * nbuf]
        wid = lax.axis_index("s") * _NC + lax.axis_index("c")
        wbase = wid * rows_per_w

        def load(c, p):
            base = wbase + c * r
            pltpu.async_copy(idx_hbm.at[pl.ds(base, r)], idx_v[p], s_in[p])
            pltpu.async_copy(x_hbm.at[pl.ds(base, r)], rows_v[p], s_in[p])

        def wait_load(p):
            pltpu.make_async_copy(idx_hbm.at[pl.ds(0, r)], idx_v[p], s_in[p]).wait()
            pltpu.make_async_copy(x_hbm.at[pl.ds(0, r)], rows_v[p], s_in[p]).wait()

        def store(c, p):
            base = wbase + c * r
            pltpu.async_copy(rows_v[p], out_hbm.at[pl.ds(base, r)], s_o[p])

        def wait_store(p):
            pltpu.make_async_copy(rows_v[p], out_hbm.at[pl.ds(0, r)], s_o[p]).wait()

        def step(c, p, drain_store, prefetch):
            wait_load(p)
            gd = pltpu.async_copy(table_hbm.at[idx_v[p]], rows_v[p],
                                  s_g[p], add=True)
            gd.wait()
            store(c, p)
            q = (p + la) % nbuf
            if drain_store:
                wait_store(q)
            if prefetch:
                load(c + la, q)

        for p in range(la):
            load(p, p)
        for p in range(nbuf):
            step(p, p, drain_store=(p + la >= nbuf), prefetch=True)

        def round_body(g, carry):
            c0 = g * nbuf
            for p in range(nbuf):
                step(c0 + p, p, drain_store=True, prefetch=True)
            return carry

        lax.fori_loop(1, rounds - 1, round_body, 0)

        c0 = (rounds - 1) * nbuf
        for p in range(nbuf):
            keep = p + la < nbuf
            step(c0 + p, p, drain_store=keep, prefetch=keep)
        for p in range(nbuf):
            wait_store(p)

    return k(x_flat, labels_flat, table)


def kernel(x, cluster_labels, table):
    b, lp1, d = x.shape
    zeros_col = jnp.zeros((b, 1), dtype=cluster_labels.dtype)
    labels = jnp.concatenate([zeros_col, cluster_labels], axis=1).reshape(-1)
    out = _sc_gather_add(x.reshape(-1, d), labels, table)
    return out.reshape(b, lp1, d)
